# baseline (speedup 1.0000x reference)
import jax
import jax.numpy as jnp
from jax import lax
from jax.experimental import pallas as pl
from jax.experimental.pallas import tpu as pltpu
from jax.experimental.pallas import tpu_sc as plsc

NUM_EMB = 100000
DIM = 128
SEQS = 4096
SEQ_LEN = 50
BATCH = SEQS * SEQ_LEN
NUM_CORES = 2
NUM_SUBCORES = 16
NUM_WORKERS = NUM_CORES * NUM_SUBCORES
B_PER_W = BATCH // NUM_WORKERS
CHUNK = 256
N_CHUNKS = B_PER_W // CHUNK
NBUF = 3


def _emb_body(table_hbm, idx_hbm, out_hbm,
              idx_all, rows0, rows1, rows2, g0, g1, g2, s0, s1, s2):
    wid = lax.axis_index("s") * NUM_CORES + lax.axis_index("c")
    base = wid * B_PER_W

    rows = (rows0, rows1, rows2)
    gsem = (g0, g1, g2)
    ssem = (s0, s1, s2)

    pltpu.sync_copy(idx_hbm.at[pl.ds(base, B_PER_W)], idx_all)

    def gather(i):
        b = i % NBUF
        return pltpu.async_copy(
            table_hbm.at[idx_all.at[pl.ds(i * CHUNK, CHUNK)]], rows[b], gsem[b])

    gath = [None] * N_CHUNKS
    scat = [None] * N_CHUNKS
    gath[0] = gather(0)
    gath[1] = gather(1)
    for i in range(N_CHUNKS):
        b = i % NBUF
        gath[i].wait()
        scat[i] = pltpu.async_copy(
            rows[b], out_hbm.at[pl.ds(base + i * CHUNK, CHUNK)], ssem[b])
        if i + 2 < N_CHUNKS:
            if i >= 1:
                scat[i - 1].wait()
            gath[i + 2] = gather(i + 2)
    scat[N_CHUNKS - 2].wait()
    scat[N_CHUNKS - 1].wait()


@jax.jit
def _embed(table, idx):
    mesh = plsc.VectorSubcoreMesh(core_axis_name="c", subcore_axis_name="s")
    return pl.kernel(
        _emb_body,
        mesh=mesh,
        out_type=jax.ShapeDtypeStruct((BATCH, DIM), jnp.float32),
        scratch_types=[
            pltpu.VMEM((B_PER_W,), jnp.int32),
            pltpu.VMEM((CHUNK, DIM), jnp.float32),
            pltpu.VMEM((CHUNK, DIM), jnp.float32),
            pltpu.VMEM((CHUNK, DIM), jnp.float32),
            pltpu.SemaphoreType.DMA,
            pltpu.SemaphoreType.DMA,
            pltpu.SemaphoreType.DMA,
            pltpu.SemaphoreType.DMA,
            pltpu.SemaphoreType.DMA,
            pltpu.SemaphoreType.DMA,
        ],
    )(table, idx)


def kernel(token_ids, embedding_matrix):
    idx = token_ids.T.reshape(-1).astype(jnp.int32)
    out = _embed(embedding_matrix, idx)
    return out.reshape(SEQ_LEN, SEQS, DIM).swapaxes(0, 1)


# speedup vs baseline: 1.0170x; 1.0170x over previous
import jax
import jax.numpy as jnp
from jax import lax
from jax.experimental import pallas as pl
from jax.experimental.pallas import tpu as pltpu
from jax.experimental.pallas import tpu_sc as plsc

NUM_EMB = 100000
DIM = 128
SEQS = 4096
SEQ_LEN = 50
BATCH = SEQS * SEQ_LEN
NUM_CORES = 2
NUM_SUBCORES = 16
NUM_WORKERS = NUM_CORES * NUM_SUBCORES
B_PER_W = BATCH // NUM_WORKERS
CHUNK = 480
_SIZES = [CHUNK] * (B_PER_W // CHUNK) + (
    [B_PER_W % CHUNK] if B_PER_W % CHUNK else [])
_OFFS = [sum(_SIZES[:i]) for i in range(len(_SIZES))]
N_CHUNKS = len(_SIZES)


def _emb_body(table_hbm, idx_hbm, out_hbm,
              idx_all, rows0, rows1, g0, g1, s0, s1):
    wid = lax.axis_index("s") * NUM_CORES + lax.axis_index("c")
    base = wid * B_PER_W

    rows = (rows0, rows1)
    gsem = (g0, g1)
    ssem = (s0, s1)

    pltpu.sync_copy(idx_hbm.at[pl.ds(base, B_PER_W)], idx_all)

    def gather(i, b):
        n = _SIZES[i]
        dst = rows[b] if n == CHUNK else rows[b].at[pl.ds(0, n)]
        return pltpu.async_copy(
            table_hbm.at[idx_all.at[pl.ds(_OFFS[i], n)]], dst, gsem[b])

    def scatter(i, b):
        n = _SIZES[i]
        src = rows[b] if n == CHUNK else rows[b].at[pl.ds(0, n)]
        return pltpu.async_copy(
            src, out_hbm.at[pl.ds(base + _OFFS[i], n)], ssem[b])

    gath = gather(0, 0)
    scat = [None, None]
    for i in range(N_CHUNKS):
        b = i % 2
        nb = 1 - b
        gath.wait()
        scat[b] = scatter(i, b)
        if i + 1 < N_CHUNKS:
            if scat[nb] is not None:
                scat[nb].wait()
            gath = gather(i + 1, nb)
    scat[0].wait()
    scat[1].wait()


@jax.jit
def _embed(table, idx):
    mesh = plsc.VectorSubcoreMesh(core_axis_name="c", subcore_axis_name="s")
    return pl.kernel(
        _emb_body,
        mesh=mesh,
        out_type=jax.ShapeDtypeStruct((BATCH, DIM), jnp.float32),
        scratch_types=[
            pltpu.VMEM((B_PER_W,), jnp.int32),
            pltpu.VMEM((CHUNK, DIM), jnp.float32),
            pltpu.VMEM((CHUNK, DIM), jnp.float32),
            pltpu.SemaphoreType.DMA,
            pltpu.SemaphoreType.DMA,
            pltpu.SemaphoreType.DMA,
            pltpu.SemaphoreType.DMA,
        ],
    )(table, idx)


def kernel(token_ids, embedding_matrix):
    idx = token_ids.T.reshape(-1).astype(jnp.int32)
    out = _embed(embedding_matrix, idx)
    return out.reshape(SEQ_LEN, SEQS, DIM).swapaxes(0, 1)


# speedup vs baseline: 1.0220x; 1.0049x over previous
import jax
import jax.numpy as jnp
from jax import lax
from jax.experimental import pallas as pl
from jax.experimental.pallas import tpu as pltpu
from jax.experimental.pallas import tpu_sc as plsc

NUM_EMB = 100000
DIM = 128
SEQS = 4096
SEQ_LEN = 50
BATCH = SEQS * SEQ_LEN
NUM_CORES = 2
NUM_SUBCORES = 16
NUM_WORKERS = NUM_CORES * NUM_SUBCORES
B_PER_W = BATCH // NUM_WORKERS
CHUNK = 400
_SIZES = [CHUNK] * (B_PER_W // CHUNK) + (
    [B_PER_W % CHUNK] if B_PER_W % CHUNK else [])
_OFFS = [sum(_SIZES[:i]) for i in range(len(_SIZES))]
N_CHUNKS = len(_SIZES)


def _emb_body(table_hbm, idx_hbm, out_hbm,
              idx_all, rows0, rows1, g0, g1, s0, s1):
    wid = lax.axis_index("s") * NUM_CORES + lax.axis_index("c")
    base = wid * B_PER_W

    rows = (rows0, rows1)
    gsem = (g0, g1)
    ssem = (s0, s1)

    pltpu.sync_copy(idx_hbm.at[pl.ds(base, B_PER_W)], idx_all)

    def gather(i, b):
        n = _SIZES[i]
        dst = rows[b] if n == CHUNK else rows[b].at[pl.ds(0, n)]
        return pltpu.async_copy(
            table_hbm.at[idx_all.at[pl.ds(_OFFS[i], n)]], dst, gsem[b])

    def scatter(i, b):
        n = _SIZES[i]
        src = rows[b] if n == CHUNK else rows[b].at[pl.ds(0, n)]
        return pltpu.async_copy(
            src, out_hbm.at[pl.ds(base + _OFFS[i], n)], ssem[b])

    gath = gather(0, 0)
    scat = [None, None]
    for i in range(N_CHUNKS):
        b = i % 2
        nb = 1 - b
        gath.wait()
        scat[b] = scatter(i, b)
        if i + 1 < N_CHUNKS:
            if scat[nb] is not None:
                scat[nb].wait()
            gath = gather(i + 1, nb)
    scat[0].wait()
    scat[1].wait()


@jax.jit
def _embed(table, idx):
    mesh = plsc.VectorSubcoreMesh(core_axis_name="c", subcore_axis_name="s")
    return pl.kernel(
        _emb_body,
        mesh=mesh,
        out_type=jax.ShapeDtypeStruct((BATCH, DIM), jnp.float32),
        scratch_types=[
            pltpu.VMEM((B_PER_W,), jnp.int32),
            pltpu.VMEM((CHUNK, DIM), jnp.float32),
            pltpu.VMEM((CHUNK, DIM), jnp.float32),
            pltpu.SemaphoreType.DMA,
            pltpu.SemaphoreType.DMA,
            pltpu.SemaphoreType.DMA,
            pltpu.SemaphoreType.DMA,
        ],
    )(table, idx)


def kernel(token_ids, embedding_matrix):
    idx = token_ids.T.reshape(-1).astype(jnp.int32)
    out = _embed(embedding_matrix, idx)
    return out.reshape(SEQ_LEN, SEQS, DIM).swapaxes(0, 1)
